# Initial kernel scaffold; baseline (speedup 1.0000x reference)
import jax
import jax.numpy as jnp
from jax.experimental import pallas as pl


def kernel(x):
    raise NotImplementedError("write your pallas kernel here")


# speedup vs baseline: 2.0200x; 2.0200x over previous
import jax
import jax.numpy as jnp
from jax import lax
from jax.experimental import pallas as pl

_B = 32
_T = 2048
_NM = 307

_ROT = (13, 15, 26, 6, 17, 29, 16, 24)


def _rotl(x, d):
    return (x << d) | lax.shift_right_logical(x, 32 - d)


def _threefry2x32(k0, k1, x0, x1):
    k2 = k0 ^ k1 ^ 0x1BD11BDA
    ks = (k0, k1, k2)
    x0 = x0 + k0
    x1 = x1 + k1
    for r in range(5):
        for j in range(4):
            x0 = x0 + x1
            x1 = _rotl(x1, _ROT[(r % 2) * 4 + j])
            x1 = x1 ^ x0
        x0 = x0 + ks[(r + 1) % 3]
        x1 = x1 + ks[(r + 2) % 3] + (r + 1)
    return x0, x1


def _split2(k0, k1):
    z = jnp.zeros_like(k0)
    a0, a1 = _threefry2x32(k0, k1, z, z)
    b0, b1 = _threefry2x32(k0, k1, z, jnp.ones_like(k0))
    return (a0, a1), (b0, b1)


def _row_bits(k0, k1, col):
    o0, o1 = _threefry2x32(k0, k1, jnp.zeros_like(col), col)
    return o0 ^ o1


def _partner(a, s, low):
    return jnp.where(low, jnp.roll(a, -s, axis=1), jnp.roll(a, s, axis=1))


def _bitonic_sort(key, col, val=None):
    n = key.shape[1]
    k = 2
    while k <= n:
        asc = (col & k) == 0
        s = k // 2
        while s >= 1:
            low = (col & s) == 0
            pk = _partner(key, s, low)
            take_self = (key < pk) == (low == asc)
            if val is not None:
                pv = _partner(val, s, low)
                val = jnp.where(take_self, val, pv)
            key = jnp.where(take_self, key, pk)
            s //= 2
        k *= 2
    return key, val


def _mask_body(o_ref):
    col = lax.broadcasted_iota(jnp.int32, (_B, _T), 1)
    row1 = lax.broadcasted_iota(jnp.int32, (_B, 1), 0)

    bk0, bk1 = _threefry2x32(jnp.zeros_like(row1), jnp.full_like(row1, 42),
                             jnp.zeros_like(row1), row1)

    (bk0, bk1), (s10, s11) = _split2(bk0, bk1)
    bits1 = _row_bits(s10, s11, col)
    (_, _), (s20, s21) = _split2(bk0, bk1)
    bits2 = _row_bits(s20, s21, col)

    flip = jnp.int32(-0x80000000)
    _, v1 = _bitonic_sort(bits1 ^ flip, col, col)
    _, v2 = _bitonic_sort(bits2 ^ flip, col, v1)

    packed = (v2 << 1) | jnp.where(col >= _NM, jnp.int32(1), jnp.int32(0))
    packed, _ = _bitonic_sort(packed, col)
    o_ref[...] = (packed & 1).astype(jnp.float32)


def kernel(x):
    del x
    return pl.pallas_call(
        _mask_body,
        out_shape=jax.ShapeDtypeStruct((_B, _T), jnp.float32),
    )()


# speedup vs baseline: 2.9343x; 1.4526x over previous
import jax
import jax.numpy as jnp
from jax import lax
from jax.experimental import pallas as pl

_B = 32
_T = 2048
_NM = 307

_ROT = (13, 15, 26, 6, 17, 29, 16, 24)


def _as_i32(v):
    return jnp.int32(v - 0x100000000 if v >= 0x80000000 else v)


def _rotl(x, d):
    return (x << d) | lax.shift_right_logical(x, 32 - d)


def _threefry2x32(k0, k1, x0, x1):
    k2 = k0 ^ k1 ^ 0x1BD11BDA
    ks = (k0, k1, k2)
    x0 = x0 + k0
    x1 = x1 + k1
    for r in range(5):
        for j in range(4):
            x0 = x0 + x1
            x1 = _rotl(x1, _ROT[(r % 2) * 4 + j])
            x1 = x1 ^ x0
        x0 = x0 + ks[(r + 1) % 3]
        x1 = x1 + ks[(r + 2) % 3] + (r + 1)
    return x0, x1


def _split2(k0, k1):
    z = jnp.zeros_like(k0)
    a0, a1 = _threefry2x32(k0, k1, z, z)
    b0, b1 = _threefry2x32(k0, k1, z, jnp.ones_like(k0))
    return (a0, a1), (b0, b1)


def _row_bits(k0, k1, col):
    o0, o1 = _threefry2x32(k0, k1, jnp.zeros_like(col), col)
    return o0 ^ o1


def _partner(a, s, low):
    return jnp.where(low, jnp.roll(a, -s, axis=1), jnp.roll(a, s, axis=1))


def _bitonic_sort(key, col, val=None):
    n = key.shape[1]
    k = 2
    while k <= n:
        asc = (col & k) == 0
        s = k // 2
        while s >= 1:
            low = (col & s) == 0
            pk = _partner(key, s, low)
            take_self = (key < pk) == (low == asc)
            if val is not None:
                pv = _partner(val, s, low)
                val = jnp.where(take_self, val, pv)
            key = jnp.where(take_self, key, pk)
            s //= 2
        k *= 2
    return key, val


def _mask_body(o_ref):
    col = lax.broadcasted_iota(jnp.int32, (_B, _T), 1)
    row1 = lax.broadcasted_iota(jnp.int32, (_B, 1), 0)

    bk0, bk1 = _threefry2x32(jnp.zeros_like(row1), jnp.full_like(row1, 42),
                             jnp.zeros_like(row1), row1)

    (bk0, bk1), (s10, s11) = _split2(bk0, bk1)
    bits1 = _row_bits(s10, s11, col)
    (_, _), (s20, s21) = _split2(bk0, bk1)
    bits2 = _row_bits(s20, s21, col)

    flip = jnp.int32(-0x80000000)
    _, v1 = _bitonic_sort(bits1 ^ flip, col, col)

    prefix = jnp.zeros((_B, 1), jnp.int32)
    rem = jnp.full((_B, 1), _NM, jnp.int32)
    for b in range(31, -1, -1):
        hm = _as_i32((-1 << (b + 1)) & 0xFFFFFFFF)
        bitv = _as_i32(1 << b)
        cand0 = ((bits2 & hm) == prefix) & ((bits2 & bitv) == 0)
        c0 = jnp.sum(jnp.where(cand0, 1, 0), axis=1, keepdims=True)
        take1 = rem >= c0
        prefix = jnp.where(take1, prefix | bitv, prefix)
        rem = jnp.where(take1, rem - c0, rem)

    keep = (bits2 ^ flip) >= (prefix ^ flip)

    packed = (v1 << 1) | jnp.where(keep, jnp.int32(1), jnp.int32(0))
    packed, _ = _bitonic_sort(packed, col)
    o_ref[...] = (packed & 1).astype(jnp.float32)


def kernel(x):
    del x
    return pl.pallas_call(
        _mask_body,
        out_shape=jax.ShapeDtypeStruct((_B, _T), jnp.float32),
    )()


# speedup vs baseline: 2.9650x; 1.0104x over previous
import jax
import jax.numpy as jnp
from jax import lax
from jax.experimental import pallas as pl

_B = 32
_T = 2048
_NM = 307

_ROT = (13, 15, 26, 6, 17, 29, 16, 24)


def _as_i32(v):
    return jnp.int32(v - 0x100000000 if v >= 0x80000000 else v)


def _rotl(x, d):
    return (x << d) | lax.shift_right_logical(x, 32 - d)


def _threefry2x32(k0, k1, x0, x1):
    k2 = k0 ^ k1 ^ 0x1BD11BDA
    ks = (k0, k1, k2)
    x0 = x0 + k0
    x1 = x1 + k1
    for r in range(5):
        for j in range(4):
            x0 = x0 + x1
            x1 = _rotl(x1, _ROT[(r % 2) * 4 + j])
            x1 = x1 ^ x0
        x0 = x0 + ks[(r + 1) % 3]
        x1 = x1 + ks[(r + 2) % 3] + (r + 1)
    return x0, x1


def _split2(k0, k1):
    z = jnp.zeros_like(k0)
    a0, a1 = _threefry2x32(k0, k1, z, z)
    b0, b1 = _threefry2x32(k0, k1, z, jnp.ones_like(k0))
    return (a0, a1), (b0, b1)


def _row_bits(k0, k1, col):
    o0, o1 = _threefry2x32(k0, k1, jnp.zeros_like(col), col)
    return o0 ^ o1


def _bitonic_sort(key, col, val=None):
    n = key.shape[1]
    a = 1
    while (1 << a) <= n:
        b = a - 1
        while b >= 0:
            s = 1 << b
            low_b = (col >> b) & 1
            m = -(low_b ^ ((col >> a) & 1))
            rl = jnp.roll(key, -s, axis=1)
            rr = jnp.roll(key, s, axis=1)
            lm = low_b - 1
            pk = rr ^ ((rl ^ rr) & lm)
            ak = key ^ m
            apk = pk ^ m
            if val is None:
                key = jnp.minimum(ak, apk) ^ m
            else:
                take_self = ak < apk
                pv = jnp.where(low_b == 0, jnp.roll(val, -s, axis=1),
                               jnp.roll(val, s, axis=1))
                val = jnp.where(take_self, val, pv)
                key = jnp.where(take_self, key, pk)
            b -= 1
        a += 1
    return key, val


def _mask_body(o_ref):
    col = lax.broadcasted_iota(jnp.int32, (_B, _T), 1)
    row1 = lax.broadcasted_iota(jnp.int32, (_B, 1), 0)

    bk0, bk1 = _threefry2x32(jnp.zeros_like(row1), jnp.full_like(row1, 42),
                             jnp.zeros_like(row1), row1)

    (bk0, bk1), (s10, s11) = _split2(bk0, bk1)
    bits1 = _row_bits(s10, s11, col)
    (_, _), (s20, s21) = _split2(bk0, bk1)
    bits2 = _row_bits(s20, s21, col)

    flip = jnp.int32(-0x80000000)
    _, v1 = _bitonic_sort(bits1 ^ flip, col, col)

    prefix = jnp.zeros((_B, 1), jnp.int32)
    rem = jnp.full((_B, 1), _NM, jnp.int32)
    for b in range(31, 14, -1):
        hm = _as_i32(((-1 << (b + 1)) | (1 << b)) & 0xFFFFFFFF)
        bitv = _as_i32(1 << b)
        c0 = jnp.sum(jnp.where((bits2 & hm) == prefix, 1, 0),
                     axis=1, keepdims=True)
        take1 = rem >= c0
        prefix = jnp.where(take1, prefix | bitv, prefix)
        rem = jnp.where(take1, rem - c0, rem)

    keep = (bits2 ^ flip) >= (prefix ^ flip)

    packed = (v1 << 1) | jnp.where(keep, jnp.int32(1), jnp.int32(0))
    packed, _ = _bitonic_sort(packed, col)
    o_ref[...] = (packed & 1).astype(jnp.float32)


def kernel(x):
    del x
    return pl.pallas_call(
        _mask_body,
        out_shape=jax.ShapeDtypeStruct((_B, _T), jnp.float32),
    )()


# speedup vs baseline: 3.6903x; 1.2446x over previous
import jax
import jax.numpy as jnp
from jax import lax
from jax.experimental import pallas as pl

_B = 32
_T = 2048
_NM = 307

_ROT = (13, 15, 26, 6, 17, 29, 16, 24)


def _as_i32(v):
    return jnp.int32(v - 0x100000000 if v >= 0x80000000 else v)


def _rotl(x, d):
    return (x << d) | lax.shift_right_logical(x, 32 - d)


def _threefry2x32(k0, k1, x0, x1):
    k2 = k0 ^ k1 ^ 0x1BD11BDA
    ks = (k0, k1, k2)
    x0 = x0 + k0
    x1 = x1 + k1
    for r in range(5):
        for j in range(4):
            x0 = x0 + x1
            x1 = _rotl(x1, _ROT[(r % 2) * 4 + j])
            x1 = x1 ^ x0
        x0 = x0 + ks[(r + 1) % 3]
        x1 = x1 + ks[(r + 2) % 3] + (r + 1)
    return x0, x1


def _split2(k0, k1):
    z = jnp.zeros_like(k0)
    a0, a1 = _threefry2x32(k0, k1, z, z)
    b0, b1 = _threefry2x32(k0, k1, z, jnp.ones_like(k0))
    return (a0, a1), (b0, b1)


def _row_bits(k0, k1, col):
    o0, o1 = _threefry2x32(k0, k1, jnp.zeros_like(col), col)
    return o0 ^ o1


def _bitonic_sort(key, col, val=None, phys=None):
    n = key.shape[1]
    nbits = n.bit_length() - 1
    if phys is None:
        phys = list(range(nbits))
    a = 1
    while (1 << a) <= n:
        pa = phys[a] if a < nbits else nbits
        asc_b = (col >> pa) & 1
        b = a - 1
        while b >= 0:
            s = 1 << phys[b]
            low_b = (col >> phys[b]) & 1
            m = -(low_b ^ asc_b)
            rl = jnp.roll(key, -s, axis=1)
            rr = jnp.roll(key, s, axis=1)
            lm = low_b - 1
            pk = rr ^ ((rl ^ rr) & lm)
            ak = key ^ m
            apk = pk ^ m
            if val is None:
                key = jnp.minimum(ak, apk) ^ m
            else:
                take_self = ak < apk
                pv = jnp.where(low_b == 0, jnp.roll(val, -s, axis=1),
                               jnp.roll(val, s, axis=1))
                val = jnp.where(take_self, val, pv)
                key = jnp.where(take_self, key, pk)
            b -= 1
        a += 1
    return key, val


def _mask_body(o_ref):
    col = lax.broadcasted_iota(jnp.int32, (_B, _T), 1)
    row1 = lax.broadcasted_iota(jnp.int32, (_B, 1), 0)
    colrev = jnp.zeros_like(col)
    for i in range(11):
        colrev = colrev | (((col >> i) & 1) << (10 - i))

    bk0, bk1 = _threefry2x32(jnp.zeros_like(row1), jnp.full_like(row1, 42),
                             jnp.zeros_like(row1), row1)

    (bk0, bk1), (s10, s11) = _split2(bk0, bk1)
    bits1 = _row_bits(s10, s11, colrev)
    (_, _), (s20, s21) = _split2(bk0, bk1)
    bits2 = _row_bits(s20, s21, colrev)

    flip = jnp.int32(-0x80000000)
    _, v1 = _bitonic_sort(bits1 ^ flip, col, colrev,
                          phys=[10 - j for j in range(11)])

    prefix = jnp.zeros((_B, 1), jnp.int32)
    rem = jnp.full((_B, 1), _NM, jnp.int32)
    for b in range(31, 14, -1):
        hm = _as_i32(((-1 << (b + 1)) | (1 << b)) & 0xFFFFFFFF)
        bitv = _as_i32(1 << b)
        c0 = jnp.sum(jnp.where((bits2 & hm) == prefix, 1, 0),
                     axis=1, keepdims=True)
        take1 = rem >= c0
        prefix = jnp.where(take1, prefix | bitv, prefix)
        rem = jnp.where(take1, rem - c0, rem)

    keep = (bits2 ^ flip) >= (prefix ^ flip)

    packed = (v1 << 1) | jnp.where(keep, jnp.int32(1), jnp.int32(0))
    packed, _ = _bitonic_sort(packed, col)
    o_ref[...] = (packed & 1).astype(jnp.float32)


def kernel(x):
    del x
    return pl.pallas_call(
        _mask_body,
        out_shape=jax.ShapeDtypeStruct((_B, _T), jnp.float32),
    )()


# speedup vs baseline: 4.1184x; 1.1160x over previous
import jax
import jax.numpy as jnp
from jax import lax
from jax.experimental import pallas as pl

_B = 32
_T = 2048
_NM = 307

_ROT = (13, 15, 26, 6, 17, 29, 16, 24)


def _as_i32(v):
    return jnp.int32(v - 0x100000000 if v >= 0x80000000 else v)


def _rotl(x, d):
    return (x << d) | lax.shift_right_logical(x, 32 - d)


def _threefry2x32(k0, k1, x0, x1):
    k2 = k0 ^ k1 ^ 0x1BD11BDA
    ks = (k0, k1, k2)
    x0 = x0 + k0
    x1 = x1 + k1
    for r in range(5):
        for j in range(4):
            x0 = x0 + x1
            x1 = _rotl(x1, _ROT[(r % 2) * 4 + j])
            x1 = x1 ^ x0
        x0 = x0 + ks[(r + 1) % 3]
        x1 = x1 + ks[(r + 2) % 3] + (r + 1)
    return x0, x1


def _split2(k0, k1):
    z = jnp.zeros_like(k0)
    a0, a1 = _threefry2x32(k0, k1, z, z)
    b0, b1 = _threefry2x32(k0, k1, z, jnp.ones_like(k0))
    return (a0, a1), (b0, b1)


def _row_bits(k0, k1, col):
    o0, o1 = _threefry2x32(k0, k1, jnp.zeros_like(col), col)
    return o0 ^ o1


def _bitonic_sort(key, col, val=None, phys=None):
    n = key.shape[1]
    nbits = n.bit_length() - 1
    if phys is None:
        phys = list(range(nbits))
    a = 1
    while (1 << a) <= n:
        pa = phys[a] if a < nbits else nbits
        asc_b = (col >> pa) & 1
        b = a - 1
        while b >= 0:
            s = 1 << phys[b]
            low_b = (col >> phys[b]) & 1
            m = -(low_b ^ asc_b)
            rl = jnp.roll(key, -s, axis=1)
            rr = jnp.roll(key, s, axis=1)
            lm = low_b - 1
            pk = rr ^ ((rl ^ rr) & lm)
            ak = key ^ m
            apk = pk ^ m
            if val is None:
                key = jnp.minimum(ak, apk) ^ m
            else:
                take_self = ak < apk
                pv = jnp.where(low_b == 0, jnp.roll(val, -s, axis=1),
                               jnp.roll(val, s, axis=1))
                val = jnp.where(take_self, val, pv)
                key = jnp.where(take_self, key, pk)
            b -= 1
        a += 1
    return key, val


def _mask_body(o_ref):
    col = lax.broadcasted_iota(jnp.int32, (_B, _T), 1)
    row1 = lax.broadcasted_iota(jnp.int32, (_B, 1), 0)
    colrev = jnp.zeros_like(col)
    for i in range(11):
        colrev = colrev | (((col >> i) & 1) << (10 - i))

    bk0, bk1 = _threefry2x32(jnp.zeros_like(row1), jnp.full_like(row1, 42),
                             jnp.zeros_like(row1), row1)

    (bk0, bk1), (s10, s11) = _split2(bk0, bk1)
    bits1 = _row_bits(s10, s11, colrev)
    (_, _), (s20, s21) = _split2(bk0, bk1)
    bits2 = _row_bits(s20, s21, colrev)

    flip = jnp.int32(-0x80000000)
    _, v1 = _bitonic_sort(bits1 ^ flip, col, colrev,
                          phys=[10 - j for j in range(11)])

    prefix = jnp.zeros((_B, 1), jnp.int32)
    rem = jnp.full((_B, 1), _NM, jnp.int32)
    for b in range(31, 14, -1):
        hm = _as_i32(((-1 << (b + 1)) | (1 << b)) & 0xFFFFFFFF)
        bitv = _as_i32(1 << b)
        c0 = jnp.sum(jnp.where((bits2 & hm) == prefix, 1, 0),
                     axis=1, keepdims=True)
        take1 = rem >= c0
        prefix = jnp.where(take1, prefix | bitv, prefix)
        rem = jnp.where(take1, rem - c0, rem)

    keep = (bits2 ^ flip) >= (prefix ^ flip)

    packed = (v1 << 1) | jnp.where(keep, jnp.int32(1), jnp.int32(0))
    packed, _ = _bitonic_sort(packed, col, phys=[10 - j for j in range(11)])
    for i, j in ((0, 10), (1, 9), (2, 8), (3, 7), (4, 6)):
        delta = (1 << j) - (1 << i)
        dm = -(((col >> i) ^ (col >> j)) & 1)
        bim = -((col >> i) & 1)
        rm = jnp.roll(packed, -delta, axis=1)
        rp = jnp.roll(packed, delta, axis=1)
        swapped = (rm & bim) | (rp & ~bim)
        packed = (packed & ~dm) | (swapped & dm)
    o_ref[...] = (packed & 1).astype(jnp.float32)


def kernel(x):
    del x
    return pl.pallas_call(
        _mask_body,
        out_shape=jax.ShapeDtypeStruct((_B, _T), jnp.float32),
    )()
